# baseline (speedup 1.0000x reference)
import functools

import jax
import jax.numpy as jnp
from jax import lax
from jax.experimental import pallas as pl
from jax.experimental.pallas import tpu as pltpu
from jax.experimental.pallas import tpu_sc as plsc

N = 10000
E = 160000
D_IN = 256
H = 512
D_OUT = 256
D_EDGE = 16
L = 3

_ROW_BLK = 2000
_EDGE_BLK = 4000
_CW = 128
_NCHUNK = H // _CW

_NS = 16
_EPT = E // _NS
_EB = 80
_NBLK = _EPT // _EB
_NH = 5112
_TRASH = _NH
_ACC_ROWS = 5120
_NPAD = 2 * _NH
_STRIPE = _ACC_ROWS // _NS


def _prep_h_body(h_ref, w_ref, b_ref, o0, o1, o2, o3):
    y = h_ref[...] @ w_ref[...] + b_ref[...]
    for c, o in enumerate((o0, o1, o2, o3)):
        o[...] = y[:, c * _CW:(c + 1) * _CW]


def _prep_h(h, W_in, b_in):
    cspec = pl.BlockSpec((_ROW_BLK, _CW), lambda i: (i, 0))
    return pl.pallas_call(
        _prep_h_body,
        grid=(N // _ROW_BLK,),
        in_specs=[
            pl.BlockSpec((_ROW_BLK, D_IN), lambda i: (i, 0)),
            pl.BlockSpec((D_IN, H), lambda i: (0, 0)),
            pl.BlockSpec((1, H), lambda i: (0, 0)),
        ],
        out_specs=[cspec] * _NCHUNK,
        out_shape=[jax.ShapeDtypeStruct((N, _CW), jnp.float32)] * _NCHUNK,
    )(h, W_in, b_in.reshape(1, H))


def _prep_ea_body(ea_ref, w_ref, b_ref, o0, o1, o2, o3):
    y = ea_ref[...] @ w_ref[...] + b_ref[...]
    for c, o in enumerate((o0, o1, o2, o3)):
        o[...] = y[:, c * _CW:(c + 1) * _CW]


def _prep_ea(edge_attr, W_e, b_e):
    cspec = pl.BlockSpec((_EDGE_BLK, _CW), lambda i: (i, 0))
    return pl.pallas_call(
        _prep_ea_body,
        grid=(E // _EDGE_BLK,),
        in_specs=[
            pl.BlockSpec((_EDGE_BLK, D_EDGE), lambda i: (i, 0)),
            pl.BlockSpec((D_EDGE, H), lambda i: (0, 0)),
            pl.BlockSpec((1, H), lambda i: (0, 0)),
        ],
        out_specs=[cspec] * _NCHUNK,
        out_shape=[jax.ShapeDtypeStruct((E, _CW), jnp.float32)] * _NCHUNK,
    )(edge_attr, W_e, b_e.reshape(1, H))


def _mlp_body(h0, h1, h2, h3, a0, a1, a2, a3, w0_ref, b0_ref, w1_ref, b1_ref,
              w2_ref, b2_ref, wo_ref, bo_ref, o0, o1, o2, o3):
    x = jnp.concatenate(
        [hc[...] + ac[...] for hc, ac in zip((h0, h1, h2, h3),
                                             (a0, a1, a2, a3))], axis=1)
    x = jnp.maximum(x @ w0_ref[...] + b0_ref[...], 0.0)
    x = jnp.maximum(x @ w1_ref[...] + b1_ref[...], 0.0)
    x = jnp.maximum(x @ w2_ref[...] + b2_ref[...], 0.0)
    x = jnp.maximum(x @ wo_ref[...] + bo_ref[...], 0.0)
    for c, o in enumerate((o0, o1, o2, o3)):
        o[...] = x[:, c * _CW:(c + 1) * _CW]


def _mlp(hs, aggrs, W0, b0, W1, b1, W2, b2, Wo, bo):
    wspec = pl.BlockSpec((H, H), lambda i: (0, 0))
    bspec = pl.BlockSpec((1, H), lambda i: (0, 0))
    cspec = pl.BlockSpec((_ROW_BLK, _CW), lambda i: (i, 0))
    return pl.pallas_call(
        _mlp_body,
        grid=(N // _ROW_BLK,),
        in_specs=[cspec] * 8 + [wspec, bspec] * 4,
        out_specs=[cspec] * _NCHUNK,
        out_shape=[jax.ShapeDtypeStruct((N, _CW), jnp.float32)] * _NCHUNK,
    )(*hs, *aggrs, W0, b0.reshape(1, H), W1, b1.reshape(1, H), W2,
      b2.reshape(1, H), Wo, bo.reshape(1, H))


def _proj_out_body(h0, h1, h2, h3, w_ref, b_ref, o_ref):
    x = jnp.concatenate([hc[...] for hc in (h0, h1, h2, h3)], axis=1)
    o_ref[...] = x @ w_ref[...] + b_ref[...]


def _proj_out(hs, W_out, b_out):
    cspec = pl.BlockSpec((_ROW_BLK, _CW), lambda i: (i, 0))
    return pl.pallas_call(
        _proj_out_body,
        grid=(N // _ROW_BLK,),
        in_specs=[cspec] * _NCHUNK + [
            pl.BlockSpec((H, D_OUT), lambda i: (0, 0)),
            pl.BlockSpec((1, D_OUT), lambda i: (0, 0)),
        ],
        out_specs=pl.BlockSpec((_ROW_BLK, D_OUT), lambda i: (i, 0)),
        out_shape=jax.ShapeDtypeStruct((N, D_OUT), jnp.float32),
    )(*hs, W_out, b_out.reshape(1, D_OUT))


def _edge_body(ht0, ht1, ht2, ht3, ea0, ea1, ea2, ea3, srcs, dsts,
               o0, o1, o2, o3, idx_src, idx_dst, idx_adj, gbuf, ebuf, accum,
               gsem, esem):
    sid = lax.axis_index("s")
    cid = lax.axis_index("c")
    base_row = sid * _STRIPE

    pltpu.sync_copy(srcs.at[sid], idx_src)
    pltpu.sync_copy(dsts.at[sid], idx_dst)

    zvec = jnp.zeros((16,), jnp.float32)

    def _zero_accum():
        gb = gbuf.at[0]

        def zrow(r, carry):
            for k in range(_CW // 16):
                gb[r, pl.ds(k * 16, 16)] = zvec
            return carry

        lax.fori_loop(0, _EB, zrow, 0, unroll=4)
        for i in range(_STRIPE // _EB):
            pltpu.sync_copy(gb, accum.at[pl.ds(base_row + i * _EB, _EB)])

    def _adjust_idx(half):
        lo = half * _NH

        trash = _TRASH + (lax.iota(jnp.int32, 16) & 7)

        def arow(r, carry):
            for k in range(_EB // 16):
                ix = (r, pl.ds(k * 16, 16))
                adj = idx_src[ix] - lo
                ok = (adj >= 0) & (adj < _NH)
                idx_adj[ix] = jnp.where(ok, adj, trash)
            return carry

        lax.fori_loop(0, _NBLK, arow, 0, unroll=4)

    def _gather_descr(b, s, ht, ea):
        g = pltpu.make_async_copy(ht.at[idx_dst.at[b]], gbuf.at[s],
                                  gsem.at[s])
        e = pltpu.make_async_copy(
            ea.at[pl.ds(sid * _EPT + b * _EB, _EB)], ebuf.at[s], esem.at[s])
        return g, e

    def _compute(s):
        gb = gbuf.at[s]
        eb = ebuf.at[s]

        def row(r, carry):
            for k in range(_CW // 16):
                ix = (r, pl.ds(k * 16, 16))
                gb[ix] = jnp.maximum(gb[ix] + eb[ix], 0.0)
            return carry

        lax.fori_loop(0, _EB, row, 0, unroll=2)

    def _run_pass(ht, ea, out, half):
        _adjust_idx(half)
        _zero_accum()
        plsc.subcore_barrier()

        for s in (0, 1):
            g, e = _gather_descr(s, s, ht, ea)
            g.start()
            e.start()

        def _do_block(b, s, start_next):
            g, e = _gather_descr(b, s, ht, ea)
            g.wait()
            e.wait()
            _compute(s)
            pltpu.sync_copy(gbuf.at[s], accum.at[idx_adj.at[b]], add=True)
            if start_next:
                @pl.when(b + 2 < _NBLK)
                def _():
                    gn, en = _gather_descr(b + 2, s, ht, ea)
                    gn.start()
                    en.start()

        def step(gi, carry):
            for s in (0, 1):
                _do_block(2 * gi + s, s, True)
            return carry

        lax.fori_loop(0, _NBLK // 2, step, 0)
        if _NBLK % 2:
            _do_block(_NBLK - 1, (_NBLK - 1) % 2, False)
        plsc.subcore_barrier()

        @pl.when(sid < _NS - 1)
        def _():
            pltpu.sync_copy(
                accum.at[pl.ds(base_row, _STRIPE)],
                out.at[pl.ds(half * _NH + base_row, _STRIPE)])

        @pl.when(sid == _NS - 1)
        def _():
            pltpu.sync_copy(
                accum.at[pl.ds(base_row, _NH - (_NS - 1) * _STRIPE)],
                out.at[pl.ds(half * _NH + base_row,
                             _NH - (_NS - 1) * _STRIPE)])

        plsc.subcore_barrier()

    @pl.when(cid == 0)
    def _():
        _run_pass(ht0, ea0, o0, 0)
        _run_pass(ht0, ea0, o0, 1)
        _run_pass(ht1, ea1, o1, 0)
        _run_pass(ht1, ea1, o1, 1)

    @pl.when(cid == 1)
    def _():
        _run_pass(ht2, ea2, o2, 0)
        _run_pass(ht2, ea2, o2, 1)
        _run_pass(ht3, ea3, o3, 0)
        _run_pass(ht3, ea3, o3, 1)


def _edge_pass(hs, eas, srcs, dsts):
    f = pl.kernel(
        _edge_body,
        out_type=[jax.ShapeDtypeStruct((_NPAD, _CW), jnp.float32)] * _NCHUNK,
        mesh=plsc.VectorSubcoreMesh(core_axis_name="c", subcore_axis_name="s"),
        scratch_types=[
            pltpu.VMEM((_NBLK, _EB), jnp.int32),
            pltpu.VMEM((_NBLK, _EB), jnp.int32),
            pltpu.VMEM((_NBLK, _EB), jnp.int32),
            pltpu.VMEM((2, _EB, _CW), jnp.float32),
            pltpu.VMEM((2, _EB, _CW), jnp.float32),
            pltpu.VMEM_SHARED((_ACC_ROWS, _CW), jnp.float32),
            pltpu.SemaphoreType.DMA((2,)),
            pltpu.SemaphoreType.DMA((2,)),
        ],
    )
    return f(*hs, *eas, srcs, dsts)


def kernel(h, edge_index, edge_attr, W_in, b_in, W_e, b_e, mlp_W0, mlp_b0,
           mlp_W1, mlp_b1, mlp_W2, mlp_b2, mlp_Wo, mlp_bo, W_out, b_out):
    srcs = edge_index[0].reshape(_NS, _NBLK, _EB)
    dsts = edge_index[1].reshape(_NS, _NBLK, _EB)
    hs = _prep_h(h, W_in, b_in)
    eas = _prep_ea(edge_attr, W_e, b_e)
    for i in range(L):
        aggrs = _edge_pass(hs, eas, srcs, dsts)
        hs = _mlp(hs, aggrs, mlp_W0[i], mlp_b0[i], mlp_W1[i], mlp_b1[i],
                  mlp_W2[i], mlp_b2[i], mlp_Wo[i], mlp_bo[i])
    return _proj_out(hs, W_out, b_out)


# speedup vs baseline: 2.5783x; 2.5783x over previous
import functools

import jax
import jax.numpy as jnp
from jax import lax
from jax.experimental import pallas as pl
from jax.experimental.pallas import tpu as pltpu
from jax.experimental.pallas import tpu_sc as plsc

N = 10000
E = 160000
D_IN = 256
H = 512
D_OUT = 256
D_EDGE = 16
L = 3

_ROW_BLK = 2000
_EDGE_BLK = 4000
_CW = 128
_NCHUNK = H // _CW

_NS = 16
_EPT = E // _NS
_EB = 80
_NBLK = _EPT // _EB
_NH = 5112
_TRASH = _NH
_ACC_ROWS = 5120
_NPAD = 2 * _NH
_STRIPE = _ACC_ROWS // _NS


def _prep_h_body(h_ref, w_ref, b_ref, o0, o1, o2, o3):
    y = h_ref[...] @ w_ref[...] + b_ref[...]
    for c, o in enumerate((o0, o1, o2, o3)):
        o[...] = y[:, c * _CW:(c + 1) * _CW]


def _prep_h(h, W_in, b_in):
    cspec = pl.BlockSpec((_ROW_BLK, _CW), lambda i: (i, 0))
    return pl.pallas_call(
        _prep_h_body,
        grid=(N // _ROW_BLK,),
        in_specs=[
            pl.BlockSpec((_ROW_BLK, D_IN), lambda i: (i, 0)),
            pl.BlockSpec((D_IN, H), lambda i: (0, 0)),
            pl.BlockSpec((1, H), lambda i: (0, 0)),
        ],
        out_specs=[cspec] * _NCHUNK,
        out_shape=[jax.ShapeDtypeStruct((N, _CW), jnp.float32)] * _NCHUNK,
    )(h, W_in, b_in.reshape(1, H))


def _prep_ea_body(ea_ref, w_ref, b_ref, o0, o1, o2, o3):
    y = ea_ref[...] @ w_ref[...] + b_ref[...]
    for c, o in enumerate((o0, o1, o2, o3)):
        o[...] = y[:, c * _CW:(c + 1) * _CW]


def _prep_ea(edge_attr, W_e, b_e):
    cspec = pl.BlockSpec((_EDGE_BLK, _CW), lambda i: (i, 0))
    return pl.pallas_call(
        _prep_ea_body,
        grid=(E // _EDGE_BLK,),
        in_specs=[
            pl.BlockSpec((_EDGE_BLK, D_EDGE), lambda i: (i, 0)),
            pl.BlockSpec((D_EDGE, H), lambda i: (0, 0)),
            pl.BlockSpec((1, H), lambda i: (0, 0)),
        ],
        out_specs=[cspec] * _NCHUNK,
        out_shape=[jax.ShapeDtypeStruct((E, _CW), jnp.float32)] * _NCHUNK,
    )(edge_attr, W_e, b_e.reshape(1, H))


def _mlp_body(h0, h1, h2, h3, a0, a1, a2, a3, w0_ref, b0_ref, w1_ref, b1_ref,
              w2_ref, b2_ref, wo_ref, bo_ref, o0, o1, o2, o3):
    x = jnp.concatenate(
        [hc[...] + ac[...] for hc, ac in zip((h0, h1, h2, h3),
                                             (a0, a1, a2, a3))], axis=1)
    x = jnp.maximum(x @ w0_ref[...] + b0_ref[...], 0.0)
    x = jnp.maximum(x @ w1_ref[...] + b1_ref[...], 0.0)
    x = jnp.maximum(x @ w2_ref[...] + b2_ref[...], 0.0)
    x = jnp.maximum(x @ wo_ref[...] + bo_ref[...], 0.0)
    for c, o in enumerate((o0, o1, o2, o3)):
        o[...] = x[:, c * _CW:(c + 1) * _CW]


def _mlp(hs, aggrs, W0, b0, W1, b1, W2, b2, Wo, bo):
    wspec = pl.BlockSpec((H, H), lambda i: (0, 0))
    bspec = pl.BlockSpec((1, H), lambda i: (0, 0))
    cspec = pl.BlockSpec((_ROW_BLK, _CW), lambda i: (i, 0))
    return pl.pallas_call(
        _mlp_body,
        grid=(N // _ROW_BLK,),
        in_specs=[cspec] * 8 + [wspec, bspec] * 4,
        out_specs=[cspec] * _NCHUNK,
        out_shape=[jax.ShapeDtypeStruct((N, _CW), jnp.float32)] * _NCHUNK,
    )(*hs, *aggrs, W0, b0.reshape(1, H), W1, b1.reshape(1, H), W2,
      b2.reshape(1, H), Wo, bo.reshape(1, H))


def _proj_out_body(h0, h1, h2, h3, w_ref, b_ref, o_ref):
    x = jnp.concatenate([hc[...] for hc in (h0, h1, h2, h3)], axis=1)
    o_ref[...] = x @ w_ref[...] + b_ref[...]


def _proj_out(hs, W_out, b_out):
    cspec = pl.BlockSpec((_ROW_BLK, _CW), lambda i: (i, 0))
    return pl.pallas_call(
        _proj_out_body,
        grid=(N // _ROW_BLK,),
        in_specs=[cspec] * _NCHUNK + [
            pl.BlockSpec((H, D_OUT), lambda i: (0, 0)),
            pl.BlockSpec((1, D_OUT), lambda i: (0, 0)),
        ],
        out_specs=pl.BlockSpec((_ROW_BLK, D_OUT), lambda i: (i, 0)),
        out_shape=jax.ShapeDtypeStruct((N, D_OUT), jnp.float32),
    )(*hs, W_out, b_out.reshape(1, D_OUT))


def _edge_body(ht0, ht1, ht2, ht3, ea0, ea1, ea2, ea3, srcs, dsts,
               o0, o1, o2, o3, idx_src, idx_dst, idx_adj, gbuf, ebuf,
               accum, gsem, esem):
    sid = lax.axis_index("s")
    cid = lax.axis_index("c")
    base_row = sid * _STRIPE

    pltpu.sync_copy(srcs.at[sid], idx_src)
    pltpu.sync_copy(dsts.at[sid], idx_dst)

    zvec = jnp.zeros((16,), jnp.float32)

    def _zero_accum():
        gb = gbuf.at[0]

        def zrow(r, carry):
            for k in range(_CW // 16):
                gb[r, pl.ds(k * 16, 16)] = zvec
            return carry

        lax.fori_loop(0, _EB, zrow, 0, unroll=4)
        for i in range(_STRIPE // _EB):
            pltpu.sync_copy(gb, accum.at[pl.ds(base_row + i * _EB, _EB)])

    def _adjust_idx(half):
        lo = half * _NH

        trash = _TRASH + (lax.iota(jnp.int32, 16) & 7)

        def arow(r, carry):
            for k in range(_EB // 16):
                ix = (r, pl.ds(k * 16, 16))
                adj = idx_src[ix] - lo
                ok = (adj >= 0) & (adj < _NH)
                idx_adj[ix] = jnp.where(ok, adj, trash)
            return carry

        lax.fori_loop(0, _NBLK, arow, 0, unroll=4)

    def _gather_descr(b, s, ht, ea):
        g = pltpu.make_async_copy(ht.at[idx_dst.at[b]], gbuf.at[s],
                                  gsem.at[s])
        e = pltpu.make_async_copy(
            ea.at[pl.ds(sid * _EPT + b * _EB, _EB)], ebuf.at[s], esem.at[s])
        return g, e

    def _compute(s):
        gb = gbuf.at[s]
        eb = ebuf.at[s]

        @functools.partial(plsc.parallel_loop, 0, _EB, unroll=4)
        def _(r):
            for k in range(_CW // 16):
                ix = (r, pl.ds(k * 16, 16))
                gb[ix] = jnp.maximum(gb[ix] + eb[ix], 0.0)

    def _run_pass(ht, ea, out, half):
        _adjust_idx(half)
        _zero_accum()
        plsc.subcore_barrier()

        for s in (0, 1):
            g, e = _gather_descr(s, s, ht, ea)
            g.start()
            e.start()

        def _do_block(b, s, start_next):
            g, e = _gather_descr(b, s, ht, ea)
            g.wait()
            e.wait()
            _compute(s)
            pltpu.sync_copy(gbuf.at[s], accum.at[idx_adj.at[b]], add=True)
            if start_next:
                @pl.when(b + 2 < _NBLK)
                def _():
                    gn, en = _gather_descr(b + 2, s, ht, ea)
                    gn.start()
                    en.start()

        def step(gi, carry):
            for s in (0, 1):
                _do_block(2 * gi + s, s, True)
            return carry

        lax.fori_loop(0, _NBLK // 2, step, 0)
        if _NBLK % 2:
            _do_block(_NBLK - 1, (_NBLK - 1) % 2, False)
        plsc.subcore_barrier()

        @pl.when(sid < _NS - 1)
        def _():
            pltpu.sync_copy(
                accum.at[pl.ds(base_row, _STRIPE)],
                out.at[pl.ds(half * _NH + base_row, _STRIPE)])

        @pl.when(sid == _NS - 1)
        def _():
            pltpu.sync_copy(
                accum.at[pl.ds(base_row, _NH - (_NS - 1) * _STRIPE)],
                out.at[pl.ds(half * _NH + base_row,
                             _NH - (_NS - 1) * _STRIPE)])

        plsc.subcore_barrier()

    @pl.when(cid == 0)
    def _():
        _run_pass(ht0, ea0, o0, 0)
        _run_pass(ht0, ea0, o0, 1)
        _run_pass(ht1, ea1, o1, 0)
        _run_pass(ht1, ea1, o1, 1)

    @pl.when(cid == 1)
    def _():
        _run_pass(ht2, ea2, o2, 0)
        _run_pass(ht2, ea2, o2, 1)
        _run_pass(ht3, ea3, o3, 0)
        _run_pass(ht3, ea3, o3, 1)


def _edge_pass(hs, eas, srcs, dsts):
    f = pl.kernel(
        _edge_body,
        out_type=[jax.ShapeDtypeStruct((_NPAD, _CW), jnp.float32)] * _NCHUNK,
        mesh=plsc.VectorSubcoreMesh(core_axis_name="c", subcore_axis_name="s"),
        scratch_types=[
            pltpu.VMEM((_NBLK, _EB), jnp.int32),
            pltpu.VMEM((_NBLK, _EB), jnp.int32),
            pltpu.VMEM((_NBLK, _EB), jnp.int32),
            pltpu.VMEM((2, _EB, _CW), jnp.float32),
            pltpu.VMEM((2, _EB, _CW), jnp.float32),
            pltpu.VMEM_SHARED((_ACC_ROWS, _CW), jnp.float32),
            pltpu.SemaphoreType.DMA((2,)),
            pltpu.SemaphoreType.DMA((2,)),
        ],
    )
    return f(*hs, *eas, srcs, dsts)


def kernel(h, edge_index, edge_attr, W_in, b_in, W_e, b_e, mlp_W0, mlp_b0,
           mlp_W1, mlp_b1, mlp_W2, mlp_b2, mlp_Wo, mlp_bo, W_out, b_out):
    srcs = edge_index[0].reshape(_NS, _NBLK, _EB)
    dsts = edge_index[1].reshape(_NS, _NBLK, _EB)
    hs = _prep_h(h, W_in, b_in)
    eas = _prep_ea(edge_attr, W_e, b_e)
    for i in range(L):
        aggrs = _edge_pass(hs, eas, srcs, dsts)
        hs = _mlp(hs, aggrs, mlp_W0[i], mlp_b0[i], mlp_W1[i], mlp_b1[i],
                  mlp_W2[i], mlp_b2[i], mlp_Wo[i], mlp_bo[i])
    return _proj_out(hs, W_out, b_out)


# speedup vs baseline: 2.6166x; 1.0148x over previous
import functools

import jax
import jax.numpy as jnp
from jax import lax
from jax.experimental import pallas as pl
from jax.experimental.pallas import tpu as pltpu
from jax.experimental.pallas import tpu_sc as plsc

N = 10000
E = 160000
D_IN = 256
H = 512
D_OUT = 256
D_EDGE = 16
L = 3

_ROW_BLK = 2000
_EDGE_BLK = 4000
_CW = 128
_NCHUNK = H // _CW

_NS = 16
_EPT = E // _NS
_EB = 80
_NBLK = _EPT // _EB
_NH = 5112
_TRASH = _NH
_ACC_ROWS = 5120
_NPAD = 2 * _NH
_STRIPE = _ACC_ROWS // _NS


def _prep_h_body(h_ref, w_ref, b_ref, o0, o1, o2, o3):
    y = h_ref[...] @ w_ref[...] + b_ref[...]
    for c, o in enumerate((o0, o1, o2, o3)):
        o[...] = y[:, c * _CW:(c + 1) * _CW]


def _prep_h(h, W_in, b_in):
    cspec = pl.BlockSpec((_ROW_BLK, _CW), lambda i: (i, 0))
    return pl.pallas_call(
        _prep_h_body,
        grid=(N // _ROW_BLK,),
        in_specs=[
            pl.BlockSpec((_ROW_BLK, D_IN), lambda i: (i, 0)),
            pl.BlockSpec((D_IN, H), lambda i: (0, 0)),
            pl.BlockSpec((1, H), lambda i: (0, 0)),
        ],
        out_specs=[cspec] * _NCHUNK,
        out_shape=[jax.ShapeDtypeStruct((N, _CW), jnp.float32)] * _NCHUNK,
    )(h, W_in, b_in.reshape(1, H))


def _prep_ea_body(ea_ref, w_ref, b_ref, o0, o1, o2, o3):
    y = ea_ref[...] @ w_ref[...] + b_ref[...]
    for c, o in enumerate((o0, o1, o2, o3)):
        o[...] = y[:, c * _CW:(c + 1) * _CW]


def _prep_ea(edge_attr, W_e, b_e):
    cspec = pl.BlockSpec((_EDGE_BLK, _CW), lambda i: (i, 0))
    return pl.pallas_call(
        _prep_ea_body,
        grid=(E // _EDGE_BLK,),
        in_specs=[
            pl.BlockSpec((_EDGE_BLK, D_EDGE), lambda i: (i, 0)),
            pl.BlockSpec((D_EDGE, H), lambda i: (0, 0)),
            pl.BlockSpec((1, H), lambda i: (0, 0)),
        ],
        out_specs=[cspec] * _NCHUNK,
        out_shape=[jax.ShapeDtypeStruct((E, _CW), jnp.float32)] * _NCHUNK,
    )(edge_attr, W_e, b_e.reshape(1, H))


def _mlp_body(h0, h1, h2, h3, a0, a1, a2, a3, w0_ref, b0_ref, w1_ref, b1_ref,
              w2_ref, b2_ref, wo_ref, bo_ref, o0, o1, o2, o3):
    x = jnp.concatenate(
        [hc[...] + ac[...] for hc, ac in zip((h0, h1, h2, h3),
                                             (a0, a1, a2, a3))], axis=1)
    x = jnp.maximum(x @ w0_ref[...] + b0_ref[...], 0.0)
    x = jnp.maximum(x @ w1_ref[...] + b1_ref[...], 0.0)
    x = jnp.maximum(x @ w2_ref[...] + b2_ref[...], 0.0)
    x = jnp.maximum(x @ wo_ref[...] + bo_ref[...], 0.0)
    for c, o in enumerate((o0, o1, o2, o3)):
        o[...] = x[:, c * _CW:(c + 1) * _CW]


def _mlp(hs, aggrs, W0, b0, W1, b1, W2, b2, Wo, bo):
    wspec = pl.BlockSpec((H, H), lambda i: (0, 0))
    bspec = pl.BlockSpec((1, H), lambda i: (0, 0))
    cspec = pl.BlockSpec((_ROW_BLK, _CW), lambda i: (i, 0))
    return pl.pallas_call(
        _mlp_body,
        grid=(N // _ROW_BLK,),
        in_specs=[cspec] * 8 + [wspec, bspec] * 4,
        out_specs=[cspec] * _NCHUNK,
        out_shape=[jax.ShapeDtypeStruct((N, _CW), jnp.float32)] * _NCHUNK,
    )(*hs, *aggrs, W0, b0.reshape(1, H), W1, b1.reshape(1, H), W2,
      b2.reshape(1, H), Wo, bo.reshape(1, H))


def _proj_out_body(h0, h1, h2, h3, w_ref, b_ref, o_ref):
    x = jnp.concatenate([hc[...] for hc in (h0, h1, h2, h3)], axis=1)
    o_ref[...] = x @ w_ref[...] + b_ref[...]


def _proj_out(hs, W_out, b_out):
    cspec = pl.BlockSpec((_ROW_BLK, _CW), lambda i: (i, 0))
    return pl.pallas_call(
        _proj_out_body,
        grid=(N // _ROW_BLK,),
        in_specs=[cspec] * _NCHUNK + [
            pl.BlockSpec((H, D_OUT), lambda i: (0, 0)),
            pl.BlockSpec((1, D_OUT), lambda i: (0, 0)),
        ],
        out_specs=pl.BlockSpec((_ROW_BLK, D_OUT), lambda i: (i, 0)),
        out_shape=jax.ShapeDtypeStruct((N, D_OUT), jnp.float32),
    )(*hs, W_out, b_out.reshape(1, D_OUT))


def _edge_body(ht0, ht1, ht2, ht3, ea0, ea1, ea2, ea3, srcs, dsts,
               o0, o1, o2, o3, idx_src, idx_dst, adj_stage, gbuf, ebuf, sbuf,
               accum, gsem, esem):
    sid = lax.axis_index("s")
    cid = lax.axis_index("c")
    base_row = sid * _STRIPE

    pltpu.sync_copy(srcs.at[sid], idx_src)
    pltpu.sync_copy(dsts.at[sid], idx_dst)

    zvec = jnp.zeros((16,), jnp.float32)

    def _zero_accum():
        gb = gbuf.at[0]

        def zrow(r, carry):
            for k in range(_CW // 16):
                gb[r, pl.ds(k * 16, 16)] = zvec
            return carry

        lax.fori_loop(0, _EB, zrow, 0, unroll=4)
        for i in range(_STRIPE // _EB):
            pltpu.sync_copy(gb, accum.at[pl.ds(base_row + i * _EB, _EB)])

    _trash_v = _TRASH + (lax.iota(jnp.int32, 16) & 7)

    def _make_adj(b, half):
        lo = half * _NH
        for k in range(_EB // 16):
            adj = idx_src[b, pl.ds(k * 16, 16)] - lo
            ok = (adj >= 0) & (adj < _NH)
            adj_stage[0, pl.ds(k * 16, 16)] = jnp.where(ok, adj, _trash_v)

    def _gather_descr(b, s, ht, ea):
        g = pltpu.make_async_copy(ht.at[idx_dst.at[b]], gbuf.at[s],
                                  gsem.at[s])
        e = pltpu.make_async_copy(
            ea.at[pl.ds(sid * _EPT + b * _EB, _EB)], ebuf.at[s], esem.at[s])
        return g, e

    def _compute(s):
        gb = gbuf.at[s]
        eb = ebuf.at[s]

        @functools.partial(plsc.parallel_loop, 0, _EB, unroll=4)
        def _(r):
            for k in range(_CW // 16):
                ix = (r, pl.ds(k * 16, 16))
                sbuf[ix] = jnp.maximum(gb[ix] + eb[ix], 0.0)

    def _run_pass(ht, ea, out, half):
        _zero_accum()
        plsc.subcore_barrier()

        for s in (0, 1):
            g, e = _gather_descr(s, s, ht, ea)
            g.start()
            e.start()

        def _do_block(b, s, start_next):
            g, e = _gather_descr(b, s, ht, ea)
            g.wait()
            e.wait()
            _compute(s)
            _make_adj(b, half)
            pltpu.sync_copy(sbuf, accum.at[adj_stage.at[0]], add=True)
            if start_next:
                @pl.when(b + 2 < _NBLK)
                def _():
                    gn, en = _gather_descr(b + 2, s, ht, ea)
                    gn.start()
                    en.start()

        def step(gi, carry):
            for s in (0, 1):
                _do_block(2 * gi + s, s, True)
            return carry

        lax.fori_loop(0, _NBLK // 2, step, 0)
        if _NBLK % 2:
            _do_block(_NBLK - 1, (_NBLK - 1) % 2, False)
        plsc.subcore_barrier()

        @pl.when(sid < _NS - 1)
        def _():
            pltpu.sync_copy(
                accum.at[pl.ds(base_row, _STRIPE)],
                out.at[pl.ds(half * _NH + base_row, _STRIPE)])

        @pl.when(sid == _NS - 1)
        def _():
            pltpu.sync_copy(
                accum.at[pl.ds(base_row, _NH - (_NS - 1) * _STRIPE)],
                out.at[pl.ds(half * _NH + base_row,
                             _NH - (_NS - 1) * _STRIPE)])

        plsc.subcore_barrier()

    @pl.when(cid == 0)
    def _():
        _run_pass(ht0, ea0, o0, 0)
        _run_pass(ht0, ea0, o0, 1)
        _run_pass(ht1, ea1, o1, 0)
        _run_pass(ht1, ea1, o1, 1)

    @pl.when(cid == 1)
    def _():
        _run_pass(ht2, ea2, o2, 0)
        _run_pass(ht2, ea2, o2, 1)
        _run_pass(ht3, ea3, o3, 0)
        _run_pass(ht3, ea3, o3, 1)


def _edge_pass(hs, eas, srcs, dsts):
    f = pl.kernel(
        _edge_body,
        out_type=[jax.ShapeDtypeStruct((_NPAD, _CW), jnp.float32)] * _NCHUNK,
        mesh=plsc.VectorSubcoreMesh(core_axis_name="c", subcore_axis_name="s"),
        scratch_types=[
            pltpu.VMEM((_NBLK, _EB), jnp.int32),
            pltpu.VMEM((_NBLK, _EB), jnp.int32),
            pltpu.VMEM((8, _EB), jnp.int32),
            pltpu.VMEM((2, _EB, _CW), jnp.float32),
            pltpu.VMEM((2, _EB, _CW), jnp.float32),
            pltpu.VMEM((_EB, _CW), jnp.float32),
            pltpu.VMEM_SHARED((_ACC_ROWS, _CW), jnp.float32),
            pltpu.SemaphoreType.DMA((2,)),
            pltpu.SemaphoreType.DMA((2,)),
        ],
    )
    return f(*hs, *eas, srcs, dsts)


def kernel(h, edge_index, edge_attr, W_in, b_in, W_e, b_e, mlp_W0, mlp_b0,
           mlp_W1, mlp_b1, mlp_W2, mlp_b2, mlp_Wo, mlp_bo, W_out, b_out):
    srcs = edge_index[0].reshape(_NS, _NBLK, _EB)
    dsts = edge_index[1].reshape(_NS, _NBLK, _EB)
    hs = _prep_h(h, W_in, b_in)
    eas = _prep_ea(edge_attr, W_e, b_e)
    for i in range(L):
        aggrs = _edge_pass(hs, eas, srcs, dsts)
        hs = _mlp(hs, aggrs, mlp_W0[i], mlp_b0[i], mlp_W1[i], mlp_b1[i],
                  mlp_W2[i], mlp_b2[i], mlp_Wo[i], mlp_bo[i])
    return _proj_out(hs, W_out, b_out)
